# baseline (speedup 1.0000x reference)
import functools

import jax
import jax.numpy as jnp
from jax import lax
from jax.experimental import pallas as pl
from jax.experimental.pallas import tpu as pltpu, tpu_sc as plsc

NUM_EMBEDDINGS = 100000

NC = 2
NS = 16
NW = NC * NS
LANES = 16
CHUNK = 128


def kernel(x, table):
    B, F = x.shape
    D = table.shape[-1]
    total = B * F
    per_w = total // NW
    n_chunks = per_w // CHUNK
    assert per_w * NW == total and n_chunks * CHUNK == per_w
    assert per_w % F == 0

    x_r = x.reshape(NW, n_chunks, CHUNK)
    mesh = plsc.VectorSubcoreMesh(core_axis_name="c", subcore_axis_name="s")

    @functools.partial(
        pl.kernel,
        mesh=mesh,
        compiler_params=pltpu.CompilerParams(use_tc_tiling_on_sc=False),
        out_type=jax.ShapeDtypeStruct((total, D), jnp.float32),
        scratch_types=[
            pltpu.VMEM((n_chunks, CHUNK), jnp.int32),
            pltpu.VMEM((CHUNK, D), jnp.float32),
            pltpu.SemaphoreType.DMA,
        ],
    )
    def k(x_hbm, tab_hbm, out_hbm, idx_v, rows_v, gsem):
        wid = lax.axis_index("s") * NC + lax.axis_index("c")
        base = wid * per_w
        pltpu.sync_copy(x_hbm.at[wid], idx_v)

        lane = lax.iota(jnp.int32, LANES)

        def body(j, _):
            for i in range(CHUNK // LANES):
                p = j * CHUNK + (i * LANES) + lane
                f = lax.rem(p, jnp.int32(F))
                idx_v[j, pl.ds(i * LANES, LANES)] = (
                    idx_v[j, pl.ds(i * LANES, LANES)] + f * NUM_EMBEDDINGS
                )
            pltpu.async_copy(tab_hbm.at[idx_v.at[j]], rows_v, gsem).wait()
            pltpu.sync_copy(rows_v, out_hbm.at[pl.ds(base + j * CHUNK, CHUNK)])
            return ()

        lax.fori_loop(0, n_chunks, body, ())

    out = k(x_r, table)
    return out.reshape(B, F, D)


# speedup vs baseline: 1.0485x; 1.0485x over previous
import functools

import jax
import jax.numpy as jnp
from jax import lax
from jax.experimental import pallas as pl
from jax.experimental.pallas import tpu as pltpu, tpu_sc as plsc

NUM_EMBEDDINGS = 100000

NC = 2
NS = 16
NW = NC * NS
LANES = 16
CHUNK = 128
NB = 8
SUBV = CHUNK // LANES


def kernel(x, table):
    B, F = x.shape
    D = table.shape[-1]
    total = B * F
    per_w = total // NW
    n_chunks = per_w // CHUNK
    assert per_w * NW == total and n_chunks * CHUNK == per_w
    assert per_w % F == 0
    assert n_chunks % NB == 0

    x_r = x.reshape(NW, n_chunks, CHUNK)
    mesh = plsc.VectorSubcoreMesh(core_axis_name="c", subcore_axis_name="s")

    @functools.partial(
        pl.kernel,
        mesh=mesh,
        compiler_params=pltpu.CompilerParams(use_tc_tiling_on_sc=False),
        out_type=jax.ShapeDtypeStruct((total, D), jnp.float32),
        scratch_types=[
            pltpu.VMEM((n_chunks, CHUNK), jnp.int32),
            pltpu.VMEM((NB, CHUNK, D), jnp.float32),
            pltpu.SemaphoreType.DMA((NB,)),
            pltpu.SemaphoreType.DMA((NB,)),
        ],
    )
    def k(x_hbm, tab_hbm, out_hbm, idx_v, rows_v, gsem, ssem):
        wid = lax.axis_index("s") * NC + lax.axis_index("c")
        base = wid * per_w
        pltpu.sync_copy(x_hbm.at[wid], idx_v)

        lane = lax.iota(jnp.int32, LANES)
        wrap = jnp.int32(F)

        def adjust(j, f_vec):
            for i in range(SUBV):
                sl = pl.ds(i * LANES, LANES)
                idx_v[j, sl] = idx_v[j, sl] + f_vec * NUM_EMBEDDINGS
                t = f_vec + LANES
                f_vec = lax.select(t >= wrap, t - wrap, t)
            return f_vec

        def fire_gather(j, b):
            pltpu.async_copy(tab_hbm.at[idx_v.at[j]], rows_v.at[b], gsem.at[b])

        def fire_scatter(j, b):
            pltpu.async_copy(
                rows_v.at[b], out_hbm.at[pl.ds(base + j * CHUNK, CHUNK)],
                ssem.at[b])

        def wait_gather(j, b):
            pltpu.make_async_copy(
                tab_hbm.at[idx_v.at[j]], rows_v.at[b], gsem.at[b]).wait()

        def wait_scatter(j, b):
            pltpu.make_async_copy(
                rows_v.at[b], out_hbm.at[pl.ds(base + j * CHUNK, CHUNK)],
                ssem.at[b]).wait()

        f_vec = lane
        for b in range(NB):
            f_vec = adjust(b, f_vec)
            fire_gather(b, b)

        def body(j0, f_vec):
            for b in range(NB):
                wait_gather(j0 + b, b)
                fire_scatter(j0 + b, b)
            for b in range(NB):
                j1 = j0 + NB + b
                wait_scatter(j0 + b, b)
                f_vec = adjust_guarded(j1, b, f_vec)
            return f_vec

        def adjust_guarded(j1, b, f_vec):
            @pl.when(j1 < n_chunks)
            def _():
                f = f_vec
                for i in range(SUBV):
                    sl = pl.ds(i * LANES, LANES)
                    idx_v[j1, sl] = idx_v[j1, sl] + f * NUM_EMBEDDINGS
                    t = f + LANES
                    f = lax.select(t >= wrap, t - wrap, t)
                fire_gather(j1, b)

            f = f_vec
            for _ in range(SUBV):
                t = f + LANES
                f = lax.select(t >= wrap, t - wrap, t)
            return f

        lax.fori_loop(0, n_chunks // NB, lambda i, fv: body(i * NB, fv), f_vec)

    out = k(x_r, table)
    return out.reshape(B, F, D)


# speedup vs baseline: 1.0490x; 1.0005x over previous
import functools

import jax
import jax.numpy as jnp
from jax import lax
from jax.experimental import pallas as pl
from jax.experimental.pallas import tpu as pltpu, tpu_sc as plsc

NUM_EMBEDDINGS = 100000

NC = 2
NS = 16
NW = NC * NS
LANES = 16
CHUNK = 128
GRP = 8
SUBV = CHUNK // LANES


def kernel(x, table):
    B, F = x.shape
    D = table.shape[-1]
    total = B * F
    per_w = total // NW
    n_chunks = per_w // CHUNK
    n_grp = n_chunks // GRP
    assert per_w * NW == total and n_grp * GRP == n_chunks
    assert per_w % F == 0

    rows_g = GRP * CHUNK
    x_r = x.reshape(NW, n_grp, rows_g)
    mesh = plsc.VectorSubcoreMesh(core_axis_name="c", subcore_axis_name="s")

    @functools.partial(
        pl.kernel,
        mesh=mesh,
        compiler_params=pltpu.CompilerParams(use_tc_tiling_on_sc=False),
        out_type=jax.ShapeDtypeStruct((NW, n_grp, rows_g, D), jnp.float32),
        scratch_types=[
            pltpu.VMEM((n_grp, rows_g), jnp.int32),
            pltpu.VMEM((2, rows_g, D), jnp.float32),
            pltpu.SemaphoreType.DMA((2,)),
            pltpu.SemaphoreType.DMA((2,)),
        ],
    )
    def k(x_hbm, tab_hbm, out_hbm, idx_v, rows_v, gsem, ssem):
        wid = lax.axis_index("s") * NC + lax.axis_index("c")
        pltpu.sync_copy(x_hbm.at[wid], idx_v)

        lane = lax.iota(jnp.int32, LANES)
        wrap = jnp.int32(F)

        def adj_body(g, f_vec):
            for i in range(GRP * SUBV):
                sl = pl.ds(i * LANES, LANES)
                idx_v[g, sl] = idx_v[g, sl] + f_vec * NUM_EMBEDDINGS
                t = f_vec + LANES
                f_vec = lax.select(t >= wrap, t - wrap, t)
            return f_vec

        lax.fori_loop(0, n_grp, adj_body, lane)

        def fire_gather(g, b):
            pltpu.async_copy(
                tab_hbm.at[idx_v.at[g]],
                rows_v.at[b], gsem.at[b])

        def wait_gather(g, b):
            pltpu.make_async_copy(
                tab_hbm.at[idx_v.at[g]],
                rows_v.at[b], gsem.at[b]).wait()

        def fire_scatter(g, b):
            pltpu.async_copy(rows_v.at[b], out_hbm.at[wid, g], ssem.at[b])

        def wait_scatter(g, b):
            pltpu.make_async_copy(
                rows_v.at[b], out_hbm.at[wid, g], ssem.at[b]).wait()

        fire_gather(0, 0)

        def body(g, _):
            b = lax.rem(g, 2)

            @pl.when(g + 1 < n_grp)
            def _():
                @pl.when(g >= 1)
                def _():
                    wait_scatter(g - 1, 1 - b)
                fire_gather(g + 1, 1 - b)

            wait_gather(g, b)
            fire_scatter(g, b)
            return ()

        lax.fori_loop(0, n_grp, body, ())
        wait_scatter(n_grp - 2, (n_grp - 2) % 2)
        wait_scatter(n_grp - 1, (n_grp - 1) % 2)

    out = k(x_r, table)
    return out.reshape(B, F, D)
